# baseline (speedup 1.0000x reference)
import functools

import jax
import jax.numpy as jnp
from jax import lax
from jax.experimental import pallas as pl
from jax.experimental.pallas import tpu as pltpu
from jax.experimental.pallas import tpu_sc as plsc

MAX_POS = 1000000
EMBED = 64

B, T = 4096, 200
N = B * T
NC, NS = 2, 16
NW = NC * NS
PER_W = N // NW
IDXROW = 128
CR = 4
CHUNK = CR * IDXROW
NCHUNKS = PER_W // CHUNK
ROWS_PER_W = PER_W // IDXROW


def _make_sc_gather():
    mesh = plsc.VectorSubcoreMesh(core_axis_name="c", subcore_axis_name="s")

    @functools.partial(
        pl.kernel,
        mesh=mesh,
        out_type=jax.ShapeDtypeStruct((N, EMBED), jnp.float32),
        compiler_params=pltpu.CompilerParams(use_tc_tiling_on_sc=False),
        scratch_types=[
            pltpu.VMEM((CR, IDXROW), jnp.int32),
            pltpu.VMEM((CHUNK, EMBED), jnp.float32),
            pltpu.SemaphoreType.DMA,
        ],
    )
    def k(tbl_hbm, idx_hbm, out_hbm, idx_v, rows_v, sem):
        wid = lax.axis_index("s") * NC + lax.axis_index("c")
        row0 = wid * ROWS_PER_W

        def chunk_body(i, carry):
            r = row0 + i * CR
            pltpu.sync_copy(idx_hbm.at[pl.ds(r, CR)], idx_v)
            copies = [
                pltpu.async_copy(
                    tbl_hbm.at[idx_v.at[j]],
                    rows_v.at[pl.ds(j * IDXROW, IDXROW)],
                    sem,
                )
                for j in range(CR)
            ]
            for c in copies:
                c.wait()
            pltpu.sync_copy(rows_v, out_hbm.at[pl.ds(r * IDXROW, CHUNK)])
            return carry

        lax.fori_loop(0, NCHUNKS, chunk_body, 0)

    return k


_sc_gather = _make_sc_gather()


def kernel(x, positional_encoding):
    idx2d = x.reshape(N // IDXROW, IDXROW)
    out = _sc_gather(positional_encoding, idx2d)
    return out.reshape(B, T, EMBED)


# speedup vs baseline: 1.0428x; 1.0428x over previous
import functools

import jax
import jax.numpy as jnp
from jax import lax
from jax.experimental import pallas as pl
from jax.experimental.pallas import tpu as pltpu
from jax.experimental.pallas import tpu_sc as plsc

MAX_POS = 1000000
EMBED = 64

B, T = 4096, 200
N = B * T
NC, NS = 2, 16
NW = NC * NS
PER_W = N // NW
IDXROW = 128
CR = 4
CHUNK = CR * IDXROW
NCHUNKS = PER_W // CHUNK
ROWS_PER_W = PER_W // IDXROW


def _make_sc_gather():
    mesh = plsc.VectorSubcoreMesh(core_axis_name="c", subcore_axis_name="s")

    @functools.partial(
        pl.kernel,
        mesh=mesh,
        out_type=jax.ShapeDtypeStruct((N, EMBED), jnp.float32),
        compiler_params=pltpu.CompilerParams(use_tc_tiling_on_sc=False),
        scratch_types=[
            pltpu.VMEM((ROWS_PER_W, IDXROW), jnp.int32),
            pltpu.VMEM((CHUNK, EMBED), jnp.float32),
            pltpu.VMEM((CHUNK, EMBED), jnp.float32),
            pltpu.SemaphoreType.DMA,
            pltpu.SemaphoreType.DMA,
            pltpu.SemaphoreType.DMA,
        ],
    )
    def k(tbl_hbm, idx_hbm, out_hbm, idx_v, rows0, rows1, g0, g1, osem):
        wid = lax.axis_index("s") * NC + lax.axis_index("c")
        row0 = wid * ROWS_PER_W
        out0 = wid * PER_W
        rows = (rows0, rows1)
        gsem = (g0, g1)

        pltpu.sync_copy(idx_hbm.at[pl.ds(row0, ROWS_PER_W)], idx_v)

        def fire_gathers(ci, buf, sem):
            return [
                pltpu.async_copy(
                    tbl_hbm.at[idx_v.at[ci * CR + j]],
                    buf.at[pl.ds(j * IDXROW, IDXROW)],
                    sem,
                )
                for j in range(CR)
            ]

        def fire_store(ci, buf):
            return pltpu.async_copy(
                buf, out_hbm.at[pl.ds(out0 + ci * CHUNK, CHUNK)], osem
            )

        def wait_gathers(buf, sem):
            for j in range(CR):
                pltpu.make_async_copy(
                    tbl_hbm.at[idx_v.at[j]],
                    buf.at[pl.ds(j * IDXROW, IDXROW)],
                    sem,
                ).wait()

        def wait_store(ci, buf):
            pltpu.make_async_copy(
                buf, out_hbm.at[pl.ds(out0 + ci * CHUNK, CHUNK)], osem
            ).wait()

        fire_gathers(0, rows[0], gsem[0])
        fire_gathers(1, rows[1], gsem[1])
        wait_gathers(rows[0], gsem[0])
        fire_store(0, rows[0])

        @pl.loop(1, NCHUNKS - 1, step=2)
        def _steady(g):
            for db in range(2):
                i = g + db
                b = (1 + db) % 2
                nb = 1 - b
                wait_store(i - 1, rows[nb])
                fire_gathers(i + 1, rows[nb], gsem[nb])
                wait_gathers(rows[b], gsem[b])
                fire_store(i, rows[b])

        wait_store(NCHUNKS - 2, rows[0])
        wait_gathers(rows[1], gsem[1])
        fire_store(NCHUNKS - 1, rows[1])
        wait_store(NCHUNKS - 1, rows[1])

    return k


_sc_gather = _make_sc_gather()


def kernel(x, positional_encoding):
    idx2d = x.reshape(N // IDXROW, IDXROW)
    out = _sc_gather(positional_encoding, idx2d)
    return out.reshape(B, T, EMBED)


# speedup vs baseline: 1.0434x; 1.0006x over previous
import functools

import jax
import jax.numpy as jnp
from jax import lax
from jax.experimental import pallas as pl
from jax.experimental.pallas import tpu as pltpu
from jax.experimental.pallas import tpu_sc as plsc

MAX_POS = 1000000
EMBED = 64

B, T = 4096, 200
NC, NS = 2, 16
NW = NC * NS
ROWS_W = B // NW
CR = 2
NCHUNKS = ROWS_W // CR
SPANS = ((0, 104), (104, 96))


def _make_sc_gather():
    mesh = plsc.VectorSubcoreMesh(core_axis_name="c", subcore_axis_name="s")

    @functools.partial(
        pl.kernel,
        mesh=mesh,
        out_type=jax.ShapeDtypeStruct((B, T, EMBED), jnp.float32),
        compiler_params=pltpu.CompilerParams(use_tc_tiling_on_sc=False),
        scratch_types=[
            pltpu.VMEM((ROWS_W, T), jnp.int32),
            pltpu.VMEM((CR, T, EMBED), jnp.float32),
            pltpu.VMEM((CR, T, EMBED), jnp.float32),
            pltpu.SemaphoreType.DMA,
            pltpu.SemaphoreType.DMA,
            pltpu.SemaphoreType.DMA,
        ],
    )
    def k(tbl_hbm, idx_hbm, out_hbm, idx_v, rows0, rows1, g0, g1, osem):
        wid = lax.axis_index("s") * NC + lax.axis_index("c")
        row0 = wid * ROWS_W
        rows = (rows0, rows1)
        gsem = (g0, g1)

        pltpu.sync_copy(idx_hbm.at[pl.ds(row0, ROWS_W)], idx_v)

        def fire_gathers(ci, buf, sem):
            for dr in range(CR):
                for off, ln in SPANS:
                    pltpu.async_copy(
                        tbl_hbm.at[idx_v.at[ci * CR + dr, pl.ds(off, ln)]],
                        buf.at[dr, pl.ds(off, ln)],
                        sem,
                    )

        def wait_gathers(buf, sem):
            for dr in range(CR):
                for off, ln in SPANS:
                    pltpu.make_async_copy(
                        tbl_hbm.at[idx_v.at[dr, pl.ds(off, ln)]],
                        buf.at[dr, pl.ds(off, ln)],
                        sem,
                    ).wait()

        def fire_store(ci, buf):
            pltpu.async_copy(
                buf, out_hbm.at[pl.ds(row0 + ci * CR, CR)], osem
            )

        def wait_store(ci, buf):
            pltpu.make_async_copy(
                buf, out_hbm.at[pl.ds(row0 + ci * CR, CR)], osem
            ).wait()

        fire_gathers(0, rows[0], gsem[0])
        fire_gathers(1, rows[1], gsem[1])
        wait_gathers(rows[0], gsem[0])
        fire_store(0, rows[0])

        @pl.loop(1, NCHUNKS - 1, step=2)
        def _steady(g):
            for db in range(2):
                i = g + db
                b = (1 + db) % 2
                nb = 1 - b
                wait_store(i - 1, rows[nb])
                fire_gathers(i + 1, rows[nb], gsem[nb])
                wait_gathers(rows[b], gsem[b])
                fire_store(i, rows[b])

        wait_store(NCHUNKS - 2, rows[0])
        wait_gathers(rows[1], gsem[1])
        fire_store(NCHUNKS - 1, rows[1])
        wait_store(NCHUNKS - 1, rows[1])

    return k


_sc_gather = _make_sc_gather()


def kernel(x, positional_encoding):
    return _sc_gather(positional_encoding, x)
